# baseline (speedup 1.0000x reference)
import functools

import jax
import jax.numpy as jnp
from jax import lax
from jax.experimental import pallas as pl
from jax.experimental.pallas import tpu as pltpu
from jax.experimental.pallas import tpu_sc as plsc

_B = 32
_N = 100000
_L = 16
_CHUNK = 10000
_NCHUNK = _N // _CHUNK
_BIG = 1e10


def _argmax_update(vals, idx, best_v, best_i):
    upd = vals > best_v
    return jnp.where(upd, vals, best_v), jnp.where(upd, idx, best_i)


def _finalize_argmax(best_v, best_i):
    m = jnp.float32(-_BIG)
    im = jnp.int32(_N)
    for l in range(_L):
        v = best_v[l]
        ii = best_i[l]
        take = (v > m) | ((v == m) & (ii < im))
        m = jnp.where(take, v, m)
        im = jnp.where(take, ii, im)
    return im


def _fps_body(x_hbm, out_hbm, yv, xc, zc, win1, win2, st):
    nc = 2
    b = lax.axis_index("s") * nc + lax.axis_index("c")
    lane = lax.iota(jnp.int32, _L)
    row_x = b * (3 * _N)
    row_y = row_x + _N
    row_z = row_y + _N

    pltpu.sync_copy(x_hbm.at[pl.ds(row_y, _N)], yv)

    def body_a(i, carry):
        bv, bi = carry
        vals = yv[pl.ds(i * _L, _L)]
        return _argmax_update(vals, lane + i * _L, bv, bi)

    bv0 = jnp.full((_L,), -_BIG, jnp.float32)
    bi0 = jnp.zeros((_L,), jnp.int32)
    bv, bi = lax.fori_loop(0, _N // _L, body_a, (bv0, bi0))
    i0 = _finalize_argmax(bv, bi)

    base = jnp.minimum((i0 // 8) * 8, _N - _L)
    off = i0 - base
    pltpu.sync_copy(x_hbm.at[pl.ds(row_x + base, _L)], win1)
    pltpu.sync_copy(x_hbm.at[pl.ds(row_z + base, _L)], win2)

    def _lane(v, k):
        r = v[0]
        for l in range(1, _L):
            r = jnp.where(k == l, v[l], r)
        return r

    cx = jnp.full((_L,), _lane(win1[...], off), jnp.float32)
    cz = jnp.full((_L,), _lane(win2[...], off), jnp.float32)
    cy = jnp.full((_L,), _lane(yv[pl.ds(base, _L)], off), jnp.float32)

    bv = jnp.full((_L,), -_BIG, jnp.float32)
    bi = jnp.zeros((_L,), jnp.int32)
    for j in range(_NCHUNK):
        pltpu.sync_copy(x_hbm.at[pl.ds(row_x + j * _CHUNK, _CHUNK)], xc)
        pltpu.sync_copy(x_hbm.at[pl.ds(row_z + j * _CHUNK, _CHUNK)], zc)

        def body_b(i, carry, j=j):
            bv, bi = carry
            vx = xc[pl.ds(i * _L, _L)]
            vz = zc[pl.ds(i * _L, _L)]
            vy = yv[pl.ds(j * _CHUNK + i * _L, _L)]
            dx = vx - cx
            dy = vy - cy
            dz = vz - cz
            d = jnp.minimum(dx * dx + dy * dy + dz * dz, _BIG)
            return _argmax_update(d, lane + (j * _CHUNK + i * _L), bv, bi)

        bv, bi = lax.fori_loop(0, _CHUNK // _L, body_b, (bv, bi))
    i1 = _finalize_argmax(bv, bi)

    res = jnp.where(lane == 0, i0, jnp.where(lane == 1, i1, 0))
    st[...] = res
    pltpu.sync_copy(st, out_hbm.at[b])


def kernel(xyz):
    x = xyz.reshape(_B * 3 * _N)
    mesh = plsc.VectorSubcoreMesh(core_axis_name="c", subcore_axis_name="s")
    fps = functools.partial(
        pl.kernel,
        mesh=mesh,
        out_type=jax.ShapeDtypeStruct((_B, _L), jnp.int32),
        scratch_types=[
            pltpu.VMEM((_N,), jnp.float32),
            pltpu.VMEM((_CHUNK,), jnp.float32),
            pltpu.VMEM((_CHUNK,), jnp.float32),
            pltpu.VMEM((_L,), jnp.float32),
            pltpu.VMEM((_L,), jnp.float32),
            pltpu.VMEM((_L,), jnp.int32),
        ],
    )(_fps_body)
    out = fps(x)
    return out[:, :2]


# speedup vs baseline: 3.5339x; 3.5339x over previous
import functools

import jax
import jax.numpy as jnp
from jax import lax
from jax.experimental import pallas as pl
from jax.experimental.pallas import tpu as pltpu
from jax.experimental.pallas import tpu_sc as plsc

_B = 32
_N = 100000
_L = 16
_CHUNK = 12800
_TAIL = _N - 7 * _CHUNK
_BIG = 1e10


def _argmax_update(vals, idx, best_v, best_i):
    upd = vals > best_v
    return jnp.where(upd, vals, best_v), jnp.where(upd, idx, best_i)


def _finalize_argmax(best_v, best_i):
    m = jnp.float32(-_BIG)
    im = jnp.int32(_N)
    for l in range(_L):
        v = best_v[l]
        ii = best_i[l]
        take = (v > m) | ((v == m) & (ii < im))
        m = jnp.where(take, v, m)
        im = jnp.where(take, ii, im)
    return im


def _lane(v, k):
    r = v[0]
    for l in range(1, _L):
        r = jnp.where(k == l, v[l], r)
    return r


def _chunk_sizes():
    return [_CHUNK] * 7 + [_TAIL]


def _fps_body(x_hbm, out_hbm, buf, buft, wv, st):
    nc = 2
    b = lax.axis_index("s") * nc + lax.axis_index("c")
    lane = lax.iota(jnp.int32, _L)

    bv = jnp.full((_L,), -_BIG, jnp.float32)
    bi = jnp.zeros((_L,), jnp.int32)
    off = 0
    for ch in _chunk_sizes():
        dst = buf if ch == _CHUNK else buft
        pltpu.sync_copy(x_hbm.at[b, :, pl.ds(off, ch)], dst)

        def body_a(i, carry, off=off, dst=dst):
            cbv, cbi = carry
            vals = dst[1, pl.ds(i * _L, _L)]
            return _argmax_update(vals, lane + (off + i * _L), cbv, cbi)

        bv, bi = lax.fori_loop(0, ch // _L, body_a, (bv, bi))
        off += ch
    i0 = _finalize_argmax(bv, bi)

    wbase = pl.multiple_of((i0 // 128) * 128, 128)
    pltpu.sync_copy(x_hbm.at[b, :, pl.ds(wbase, 128)], wv)
    woff = i0 - wbase
    w8 = pl.multiple_of(jnp.minimum((woff // 8) * 8, 128 - _L), 8)
    wk = woff - w8
    cx = jnp.full((_L,), _lane(wv[0, pl.ds(w8, _L)], wk), jnp.float32)
    cy = jnp.full((_L,), _lane(wv[1, pl.ds(w8, _L)], wk), jnp.float32)
    cz = jnp.full((_L,), _lane(wv[2, pl.ds(w8, _L)], wk), jnp.float32)

    bv = jnp.full((_L,), -_BIG, jnp.float32)
    bi = jnp.zeros((_L,), jnp.int32)
    off = 0
    for ch in _chunk_sizes():
        dst = buf if ch == _CHUNK else buft
        pltpu.sync_copy(x_hbm.at[b, :, pl.ds(off, ch)], dst)

        def body_b(i, carry, off=off, dst=dst):
            cbv, cbi = carry
            vx = dst[0, pl.ds(i * _L, _L)]
            vy = dst[1, pl.ds(i * _L, _L)]
            vz = dst[2, pl.ds(i * _L, _L)]
            dx = vx - cx
            dy = vy - cy
            dz = vz - cz
            d = jnp.minimum(dx * dx + dy * dy + dz * dz, _BIG)
            return _argmax_update(d, lane + (off + i * _L), cbv, cbi)

        bv, bi = lax.fori_loop(0, ch // _L, body_b, (bv, bi))
        off += ch
    i1 = _finalize_argmax(bv, bi)

    res = jnp.where(lane == 0, i0, jnp.where(lane == 1, i1, 0))
    st[...] = res.reshape(1, _L)
    pltpu.sync_copy(st, out_hbm.at[b])


def kernel(xyz):
    x = xyz.reshape(_B, 3, _N)
    mesh = plsc.VectorSubcoreMesh(core_axis_name="c", subcore_axis_name="s")
    fps = functools.partial(
        pl.kernel,
        mesh=mesh,
        out_type=jax.ShapeDtypeStruct((_B, 1, _L), jnp.int32),
        scratch_types=[
            pltpu.VMEM((3, _CHUNK), jnp.float32),
            pltpu.VMEM((3, _TAIL), jnp.float32),
            pltpu.VMEM((3, 128), jnp.float32),
            pltpu.VMEM((1, _L), jnp.int32),
        ],
    )(_fps_body)
    out = fps(x)
    return out[:, 0, :2]
